# baseline (speedup 1.0000x reference)
import functools
import jax
import jax.numpy as jnp
from jax import lax
from jax.experimental import pallas as pl
from jax.experimental.pallas import tpu as pltpu
from jax.experimental.pallas import tpu_sc as plsc

_C = 1000
_N = 4096
_M = 26
_CCHUNK = 8
_NCHUNK = _C // _CCHUNK
_ITEMS = _M * _NCHUNK


def _sc_one_hot(xt_flat):
    info = plsc.get_sparse_core_info()
    nw = info.num_cores * info.num_subcores
    n_iter = (_ITEMS + nw - 1) // nw
    mesh = plsc.VectorSubcoreMesh(core_axis_name="c", subcore_axis_name="s")

    @functools.partial(
        pl.kernel,
        out_type=jax.ShapeDtypeStruct((_M * _C, _N), jnp.int32),
        mesh=mesh,
        scratch_types=[
            pltpu.VMEM((_N,), jnp.int32),
            pltpu.VMEM((_CCHUNK, _N), jnp.int32),
        ],
    )
    def k(xt_hbm, out_hbm, row_v, buf_v):
        wid = lax.axis_index("s") * info.num_cores + lax.axis_index("c")

        def item_body(t, _):
            item = wid + t * nw

            @pl.when(item < _ITEMS)
            def _():
                j = item // _NCHUNK
                c0 = (item % _NCHUNK) * _CCHUNK
                pltpu.sync_copy(xt_hbm.at[pl.ds(j * _N, _N)], row_v)

                def seg(s, _):
                    v = row_v[pl.ds(s * 16, 16)]
                    for cc in range(_CCHUNK):
                        buf_v[cc, pl.ds(s * 16, 16)] = jnp.where(
                            v == c0 + cc, 1, 0).astype(jnp.int32)
                    return 0
                lax.fori_loop(0, _N // 16, seg, 0)
                pltpu.sync_copy(buf_v,
                                out_hbm.at[pl.ds(item * _CCHUNK, _CCHUNK)])
            return 0
        lax.fori_loop(0, n_iter, item_body, 0)

    return k(xt_flat)


def kernel(x):
    n, m = x.shape
    xt_flat = x.T.reshape(m * n)
    t2 = _sc_one_hot(xt_flat)
    out_dtype = jnp.zeros((), jnp.int64).dtype
    return jnp.transpose(t2.reshape(m, _C, n), (2, 0, 1)).astype(out_dtype)


# speedup vs baseline: 4.1465x; 4.1465x over previous
import jax
import jax.numpy as jnp
from jax.experimental import pallas as pl

_NUM_CLASSES = 1000
_LANE_CHUNK = 1024


def _one_hot_body(xt_ref, o_ref):
    i = pl.program_id(1)
    xi = xt_ref[0, 0, pl.ds(i * _LANE_CHUNK, _LANE_CHUNK)]
    cls = jax.lax.broadcasted_iota(jnp.int32, (_NUM_CLASSES, _LANE_CHUNK), 0)
    o_ref[0] = (xi[None, :] == cls).astype(o_ref.dtype)


def kernel(x):
    n, m = x.shape
    xt = x.T.reshape(m, 1, n)
    out_dtype = jnp.zeros((), jnp.int64).dtype
    t = pl.pallas_call(
        _one_hot_body,
        grid=(m, n // _LANE_CHUNK),
        in_specs=[pl.BlockSpec((1, 1, n), lambda j, i: (j, 0, 0))],
        out_specs=pl.BlockSpec((1, _NUM_CLASSES, _LANE_CHUNK),
                               lambda j, i: (j, 0, i)),
        out_shape=jax.ShapeDtypeStruct((m, _NUM_CLASSES, n), out_dtype),
    )(xt)
    return jnp.transpose(t, (2, 0, 1))
